# Initial kernel scaffold; baseline (speedup 1.0000x reference)
import jax
import jax.numpy as jnp
from jax.experimental import pallas as pl


def kernel(x, W_router, W_gate, W_up, W_down):
    raise NotImplementedError("write your pallas kernel here")


# speedup vs baseline: 3.0305x; 3.0305x over previous
import functools
import math

import jax
import jax.numpy as jnp
from jax.experimental import pallas as pl

E = 64
TOP_K = 2
D_MODEL = 768
HIDDEN = 768
T = 2048
DEPTH = 12
DEPTH_SCALE = 1.0 / math.sqrt(DEPTH)

_NEG = -1e30


def _router_body(x_ref, wr_ref, wmask_ref):
    x = x_ref[...]
    logits = jax.lax.dot_general(
        x, wr_ref[...],
        dimension_numbers=(((1,), (1,)), ((), ())),
        preferred_element_type=jnp.float32,
    )
    col = jax.lax.broadcasted_iota(jnp.int32, logits.shape, 1)
    m1 = jnp.max(logits, axis=1, keepdims=True)
    i1 = jnp.argmax(logits, axis=1)[:, None]
    masked = jnp.where(col == i1, _NEG, logits)
    m2 = jnp.max(masked, axis=1, keepdims=True)
    i2 = jnp.argmax(masked, axis=1)[:, None]
    w1 = jax.nn.sigmoid(m1 - m2)
    w2 = 1.0 - w1
    wmask = jnp.where(col == i1, w1, 0.0) + jnp.where(col == i2, w2, 0.0)
    wmask_ref[0, ...] = wmask.T


def _ffn_body(wcol_ref, x_ref, wg_ref, wu_ref, wd_ref, out_ref):
    e = pl.program_id(0)

    @pl.when(e == 0)
    def _init():
        out_ref[...] = jnp.zeros_like(out_ref)

    x = x_ref[...]
    g = jax.lax.dot_general(
        x, wg_ref[0], dimension_numbers=(((1,), (1,)), ((), ())),
        preferred_element_type=jnp.float32)
    u = jax.lax.dot_general(
        x, wu_ref[0], dimension_numbers=(((1,), (1,)), ((), ())),
        preferred_element_type=jnp.float32)
    h = (g * jax.nn.sigmoid(g)) * u
    y = jax.lax.dot_general(
        h, wd_ref[0], dimension_numbers=(((1,), (1,)), ((), ())),
        preferred_element_type=jnp.float32)
    w = wcol_ref[0, 0, :][:, None]
    out_ref[...] += (w * y) * DEPTH_SCALE


@jax.jit
def _moe(x, W_router, W_gate, W_up, W_down):
    Bx, Tx, C = x.shape
    flat_x = x.reshape(Tx, C)

    wmask = pl.pallas_call(
        _router_body,
        out_shape=jax.ShapeDtypeStruct((1, E, Tx), jnp.float32),
        in_specs=[
            pl.BlockSpec((Tx, C), lambda: (0, 0)),
            pl.BlockSpec((E, C), lambda: (0, 0)),
        ],
        out_specs=pl.BlockSpec((1, E, Tx), lambda: (0, 0, 0)),
    )(flat_x, W_router)

    wmask3 = wmask.reshape(E, 1, Tx)

    out = pl.pallas_call(
        _ffn_body,
        grid=(E,),
        out_shape=jax.ShapeDtypeStruct((Tx, C), jnp.float32),
        in_specs=[
            pl.BlockSpec((1, 1, Tx), lambda e: (e, 0, 0)),
            pl.BlockSpec((Tx, C), lambda e: (0, 0)),
            pl.BlockSpec((1, HIDDEN, C), lambda e: (e, 0, 0)),
            pl.BlockSpec((1, HIDDEN, C), lambda e: (e, 0, 0)),
            pl.BlockSpec((1, C, HIDDEN), lambda e: (e, 0, 0)),
        ],
        out_specs=pl.BlockSpec((Tx, C), lambda e: (0, 0)),
    )(wmask3, flat_x, W_gate, W_up, W_down)

    return out.reshape(Bx, Tx, C)


def kernel(x, W_router, W_gate, W_up, W_down):
    return _moe(x, W_router, W_gate, W_up, W_down)


# speedup vs baseline: 5.7472x; 1.8965x over previous
import functools
import math

import jax
from jax import lax
import jax.numpy as jnp
from jax.experimental import pallas as pl
from jax.experimental.pallas import tpu as pltpu
from jax.experimental.pallas import tpu_sc as plsc

E = 64
D = 768
H = 768
T = 2048
R = 64
NB = 128
NPAD = NB * R
DEPTH_SCALE = 1.0 / math.sqrt(12)

NC = 2
NS = 16
NW = NC * NS
CHUNK = T // NW


def _cumsum_rows(a):
    n, m = a.shape
    s = 1
    while s < n:
        shifted = jnp.concatenate([jnp.zeros((s, m), a.dtype), a[:-s]], axis=0)
        a = a + shifted
        s *= 2
    return a


def _router_body(x_ref, wr_ref, p0_ref, p1_ref, w0_ref, w1_ref, be_ref, nb_ref):
    x = x_ref[...]
    logits = jax.lax.dot_general(
        x, wr_ref[...], dimension_numbers=(((1,), (1,)), ((), ())),
        preferred_element_type=jnp.float32)
    col = jax.lax.broadcasted_iota(jnp.int32, logits.shape, 1)
    m1 = jnp.max(logits, axis=1, keepdims=True)
    i1 = jnp.argmax(logits, axis=1)[:, None]
    masked = jnp.where(col == i1, -1e30, logits)
    m2 = jnp.max(masked, axis=1, keepdims=True)
    i2 = jnp.argmax(masked, axis=1)[:, None]
    w0 = jax.nn.sigmoid(m1 - m2)
    w0_ref[...] = w0
    w1_ref[...] = 1.0 - w0

    oh1 = (col == i1).astype(jnp.float32)
    oh2 = (col == i2).astype(jnp.float32)
    c1 = _cumsum_rows(oh1)
    c2 = _cumsum_rows(oh2)
    rank0 = jnp.sum(oh1 * (c1 - 1.0 + c2), axis=1, keepdims=True)
    rank1 = jnp.sum(oh2 * (c2 - 1.0 + c1), axis=1, keepdims=True)

    counts = c1[-1:, :] + c2[-1:, :]
    nb_e = (counts + (R - 1)) // R
    rr = jax.lax.broadcasted_iota(jnp.int32, (E, E), 0)
    cc = jax.lax.broadcasted_iota(jnp.int32, (E, E), 1)
    lt = (rr < cc).astype(jnp.float32)
    bs = jax.lax.dot_general(nb_e, lt,
                             dimension_numbers=(((1,), (0,)), ((), ())),
                             preferred_element_type=jnp.float32)
    ends = bs + nb_e
    total = ends[0, E - 1]

    base0 = jnp.sum(oh1 * bs, axis=1, keepdims=True) * R
    base1 = jnp.sum(oh2 * bs, axis=1, keepdims=True) * R
    p0_ref[...] = (base0 + rank0).astype(jnp.int32)
    p1_ref[...] = (base1 + rank1).astype(jnp.int32)

    b_iota = jax.lax.broadcasted_iota(jnp.int32, (NB, E), 0).astype(jnp.float32)
    ends_b = jnp.broadcast_to(ends, (NB, E))
    be = jnp.sum((ends_b <= b_iota).astype(jnp.float32), axis=1, keepdims=True)
    be_last = jnp.sum((ends <= (total - 1.0)).astype(jnp.float32), axis=1,
                      keepdims=True)
    be = jnp.minimum(be, be_last)
    be_ref[...] = be.astype(jnp.int32).reshape(1, NB)
    nb_ref[...] = jnp.full((1, 1), total, jnp.float32).astype(jnp.int32)


def _ffn_body(be_ref, nb_ref, xs_ref, wg_ref, wu_ref, wd_ref, ys_ref):
    b = pl.program_id(0)

    @pl.when(b < nb_ref[0])
    def _():
        xb = xs_ref[...]
        g = jax.lax.dot_general(xb, wg_ref[0],
                                dimension_numbers=(((1,), (1,)), ((), ())),
                                preferred_element_type=jnp.float32)
        u = jax.lax.dot_general(xb, wu_ref[0],
                                dimension_numbers=(((1,), (1,)), ((), ())),
                                preferred_element_type=jnp.float32)
        h = (g * jax.nn.sigmoid(g)) * u
        ys_ref[...] = jax.lax.dot_general(h, wd_ref[0],
                                          dimension_numbers=(((1,), (1,)), ((), ())),
                                          preferred_element_type=jnp.float32) * DEPTH_SCALE


def _combine_body(a_ref, b_ref, w0_ref, w1_ref, out_ref):
    out_ref[...] = w0_ref[...] * a_ref[...] + w1_ref[...] * b_ref[...]


_SC_MESH = plsc.VectorSubcoreMesh(
    core_axis_name="c", subcore_axis_name="s", num_cores=NC, num_subcores=NS)


@functools.partial(
    pl.kernel,
    out_type=jax.ShapeDtypeStruct((NPAD, D), jnp.float32),
    mesh=_SC_MESH,
    scratch_types=[
        pltpu.VMEM((CHUNK, D), jnp.float32),
        pltpu.VMEM((CHUNK,), jnp.int32),
        pltpu.VMEM((CHUNK,), jnp.int32),
        pltpu.SemaphoreType.DMA,
    ],
)
def _sc_dispatch(x_hbm, p0_hbm, p1_hbm, xs_hbm, xbuf, idx0, idx1, sem):
    w = lax.axis_index("s") * NC + lax.axis_index("c")
    base = w * CHUNK
    pltpu.sync_copy(x_hbm.at[pl.ds(base, CHUNK)], xbuf)
    pltpu.sync_copy(p0_hbm.at[w], idx0)
    pltpu.sync_copy(p1_hbm.at[w], idx1)
    c0 = pltpu.async_copy(xbuf, xs_hbm.at[idx0], sem)
    c1 = pltpu.async_copy(xbuf, xs_hbm.at[idx1], sem)
    c0.wait()
    c1.wait()


@functools.partial(
    pl.kernel,
    out_type=(
        jax.ShapeDtypeStruct((T, D), jnp.float32),
        jax.ShapeDtypeStruct((T, D), jnp.float32),
    ),
    mesh=_SC_MESH,
    scratch_types=[
        pltpu.VMEM((CHUNK, D), jnp.float32),
        pltpu.VMEM((CHUNK, D), jnp.float32),
        pltpu.VMEM((CHUNK,), jnp.int32),
        pltpu.VMEM((CHUNK,), jnp.int32),
        pltpu.SemaphoreType.DMA,
    ],
)
def _sc_combine(ys_hbm, p0_hbm, p1_hbm, a_hbm, b_hbm, buf0, buf1, idx0, idx1, sem):
    w = lax.axis_index("s") * NC + lax.axis_index("c")
    base = w * CHUNK
    pltpu.sync_copy(p0_hbm.at[w], idx0)
    pltpu.sync_copy(p1_hbm.at[w], idx1)
    c0 = pltpu.async_copy(ys_hbm.at[idx0], buf0, sem)
    c1 = pltpu.async_copy(ys_hbm.at[idx1], buf1, sem)
    c0.wait()
    c1.wait()
    pltpu.sync_copy(buf0, a_hbm.at[pl.ds(base, CHUNK)])
    pltpu.sync_copy(buf1, b_hbm.at[pl.ds(base, CHUNK)])


@jax.jit
def _moe(x, W_router, W_gate, W_up, W_down):
    Bx, Tx, C = x.shape
    flat_x = x.reshape(Tx, C)

    p0, p1, w0, w1, be, nbt = pl.pallas_call(
        _router_body,
        out_shape=(
            jax.ShapeDtypeStruct((Tx, 1), jnp.int32),
            jax.ShapeDtypeStruct((Tx, 1), jnp.int32),
            jax.ShapeDtypeStruct((Tx, 1), jnp.float32),
            jax.ShapeDtypeStruct((Tx, 1), jnp.float32),
            jax.ShapeDtypeStruct((1, NB), jnp.int32),
            jax.ShapeDtypeStruct((1, 1), jnp.int32),
        ),
        in_specs=[
            pl.BlockSpec((Tx, C), lambda: (0, 0)),
            pl.BlockSpec((E, C), lambda: (0, 0)),
        ],
        out_specs=(
            pl.BlockSpec((Tx, 1), lambda: (0, 0)),
            pl.BlockSpec((Tx, 1), lambda: (0, 0)),
            pl.BlockSpec((Tx, 1), lambda: (0, 0)),
            pl.BlockSpec((Tx, 1), lambda: (0, 0)),
            pl.BlockSpec((1, NB), lambda: (0, 0)),
            pl.BlockSpec((1, 1), lambda: (0, 0)),
        ),
    )(flat_x, W_router)

    p0w = p0.reshape(NW, CHUNK)
    p1w = p1.reshape(NW, CHUNK)

    xs = _sc_dispatch(flat_x, p0w, p1w)

    ys = pl.pallas_call(
        _ffn_body,
        grid_spec=pltpu.PrefetchScalarGridSpec(
            num_scalar_prefetch=2,
            grid=(NB,),
            in_specs=[
                pl.BlockSpec((R, C), lambda b, be, nb: (b, 0)),
                pl.BlockSpec((1, H, C), lambda b, be, nb: (be[b], 0, 0)),
                pl.BlockSpec((1, H, C), lambda b, be, nb: (be[b], 0, 0)),
                pl.BlockSpec((1, C, H), lambda b, be, nb: (be[b], 0, 0)),
            ],
            out_specs=pl.BlockSpec((R, C), lambda b, be, nb: (b, 0)),
        ),
        out_shape=jax.ShapeDtypeStruct((NPAD, C), jnp.float32),
    )(be[0], nbt[0], xs, W_gate, W_up, W_down)

    a, b = _sc_combine(ys, p0w, p1w)

    out = pl.pallas_call(
        _combine_body,
        out_shape=jax.ShapeDtypeStruct((Tx, C), jnp.float32),
        in_specs=[
            pl.BlockSpec((Tx, C), lambda: (0, 0)),
            pl.BlockSpec((Tx, C), lambda: (0, 0)),
            pl.BlockSpec((Tx, 1), lambda: (0, 0)),
            pl.BlockSpec((Tx, 1), lambda: (0, 0)),
        ],
        out_specs=pl.BlockSpec((Tx, C), lambda: (0, 0)),
    )(a, b, w0, w1)
    return out.reshape(Bx, Tx, C)


def kernel(x, W_router, W_gate, W_up, W_down):
    return _moe(x, W_router, W_gate, W_up, W_down)
